# baseline (speedup 1.0000x reference)
import functools

import jax
import jax.numpy as jnp
from jax import lax
from jax.experimental import pallas as pl
from jax.experimental.pallas import tpu as pltpu
from jax.experimental.pallas import tpu_sc as plsc

_B = 16
_D = 1024


def kernel(flat, cu_seqlens):
    mesh = plsc.VectorSubcoreMesh(
        core_axis_name="c", subcore_axis_name="s", num_cores=1, num_subcores=16
    )

    @functools.partial(
        pl.kernel,
        out_type=jax.ShapeDtypeStruct((_B, _D), jnp.float32),
        mesh=mesh,
        scratch_types=[
            pltpu.VMEM((2 * _B,), jnp.int32),
            pltpu.VMEM((2,), jnp.int32),
            pltpu.VMEM((2, _D), jnp.float32),
            pltpu.SemaphoreType.DMA,
        ],
        compiler_params=pltpu.CompilerParams(
            needs_layout_passes=False,
            skip_device_barrier=True,
            disable_bounds_checks=True,
            disable_semaphore_checks=True,
        ),
    )
    def sc_kernel(flat_hbm, cu_hbm, out_hbm, cu_v, idx_v, row_v, sem):
        wid = lax.axis_index("s")

        @pl.when(wid < _B // 2)
        def _():
            pltpu.sync_copy(cu_hbm, cu_v.at[pl.ds(0, _B + 1)])
            pos = lax.broadcasted_iota(jnp.int32, (_B,), 0) + 1
            lane = pos - 1
            ends = plsc.load_gather(cu_v, [pos])
            plsc.store_scatter(idx_v, [lane & 1], ends - 1, mask=lane >> 1 == wid)
            pltpu.async_copy(flat_hbm.at[idx_v], row_v, sem).wait()
            pltpu.sync_copy(row_v, out_hbm.at[pl.ds(2 * wid, 2)])

    return sc_kernel(flat, cu_seqlens)


# speedup vs baseline: 1.0199x; 1.0199x over previous
import functools

import jax
import jax.numpy as jnp
from jax import lax
from jax.experimental import pallas as pl
from jax.experimental.pallas import tpu as pltpu
from jax.experimental.pallas import tpu_sc as plsc

_B = 16
_D = 1024


def kernel(flat, cu_seqlens):
    mesh = plsc.VectorSubcoreMesh(
        core_axis_name="c", subcore_axis_name="s", num_cores=1, num_subcores=16
    )

    @functools.partial(
        pl.kernel,
        out_type=jax.ShapeDtypeStruct((_B, _D), jnp.float32),
        mesh=mesh,
        scratch_types=[
            pltpu.VMEM((2 * _B,), jnp.int32),
            pltpu.VMEM((1,), jnp.int32),
            pltpu.VMEM((1, _D), jnp.float32),
        ],
        compiler_params=pltpu.CompilerParams(
            needs_layout_passes=False,
            skip_device_barrier=True,
            disable_bounds_checks=True,
            disable_semaphore_checks=True,
        ),
    )
    def sc_kernel(flat_hbm, cu_hbm, out_hbm, cu_v, idx_v, row_v):
        wid = lax.axis_index("s")
        pltpu.sync_copy(cu_hbm, cu_v.at[pl.ds(0, _B + 1)])
        pos = lax.broadcasted_iota(jnp.int32, (_B,), 0) + 1
        ends = plsc.load_gather(cu_v, [pos])
        zeros = jnp.zeros((_B,), jnp.int32)
        plsc.store_scatter(idx_v, [zeros], ends - 1, mask=pos - 1 == wid)
        pltpu.sync_copy(flat_hbm.at[idx_v], row_v)
        pltpu.sync_copy(row_v, out_hbm.at[pl.ds(wid, 1)])

    return sc_kernel(flat, cu_seqlens)
